# baseline (speedup 1.0000x reference)
import functools

import jax
import jax.numpy as jnp
from jax import lax
from jax.experimental import pallas as pl
from jax.experimental.pallas import tpu as pltpu
from jax.experimental.pallas import tpu_sc as plsc

_NC = 2
_NS = 16
_NW = _NC * _NS
_L = 16


def _sc_embed_gather(table, idx_flat, modality):
  V, D = table.shape
  B = idx_flat.shape[0]
  b_per_w = B // _NW
  R = 16
  NBUF = 3
  n_chunks = b_per_w // R
  mesh = plsc.VectorSubcoreMesh(core_axis_name="c", subcore_axis_name="s")

  @functools.partial(
      pl.kernel,
      mesh=mesh,
      out_type=jax.ShapeDtypeStruct((B, D), jnp.float32),
      scratch_types=[
          pltpu.VMEM((b_per_w,), jnp.int32),
          pltpu.VMEM((NBUF, R, D), jnp.float32),
          pltpu.VMEM((D,), jnp.float32),
          pltpu.SemaphoreType.DMA((NBUF,)),
          pltpu.SemaphoreType.DMA((NBUF,)),
      ],
  )
  def k(table_hbm, idx_hbm, mod_hbm, out_hbm,
        idx_v, rows_v, mod_v, gsem, osem):
    wid = lax.axis_index("s") * _NC + lax.axis_index("c")
    base = wid * b_per_w
    pltpu.sync_copy(idx_hbm.at[pl.ds(base, b_per_w)], idx_v)
    pltpu.sync_copy(mod_hbm, mod_v)

    def start_gather(c):
      b = c % NBUF
      return pltpu.async_copy(
          table_hbm.at[idx_v.at[pl.ds(c * R, R)]], rows_v.at[b],
          gsem.at[b])

    g = {c: start_gather(c) for c in range(min(NBUF - 1, n_chunks))}
    o = {}
    for c in range(n_chunks):
      b = c % NBUF
      g[c].wait()
      nxt = c + NBUF - 1
      if nxt < n_chunks:
        prev = nxt - NBUF
        if prev >= 0:
          o[prev].wait()
        g[nxt] = start_gather(nxt)

      UNROLL = 8

      def col_body(j, _):
        for u in range(UNROLL):
          sl = pl.ds((j * UNROLL + u) * _L, _L)
          m = mod_v[sl]
          for r in range(R):
            rows_v[b, r, sl] = rows_v[b, r, sl] + m
        return 0

      lax.fori_loop(0, D // (_L * UNROLL), col_body, 0)

      o[c] = pltpu.async_copy(
          rows_v.at[b], out_hbm.at[pl.ds(base + c * R, R)], osem.at[b])
    for c in range(max(0, n_chunks - NBUF), n_chunks):
      o[c].wait()

  return k(table, idx_flat, modality)


def _tc_pos_emb(cache, bs):
  S, D = cache.shape
  blk = 256

  def body(c_ref, o_ref):
    o_ref[...] = c_ref[...][None]

  return pl.pallas_call(
      body,
      grid=(bs, S // blk),
      in_specs=[pl.BlockSpec((blk, D), lambda b, i: (i, 0))],
      out_specs=pl.BlockSpec((1, blk, D), lambda b, i: (b, i, 0)),
      out_shape=jax.ShapeDtypeStruct((bs, S, D), jnp.float32),
  )(cache)


def _tc_ones(rows, S):
  blk = 512

  def body(o_ref):
    o_ref[...] = jnp.ones_like(o_ref)

  return pl.pallas_call(
      body,
      grid=(rows, S // blk),
      out_specs=pl.BlockSpec((1, blk, S), lambda i, j: (i, j, 0)),
      out_shape=jax.ShapeDtypeStruct((rows, S, S), jnp.float32),
  )()


def kernel(inputs, shared_embed_weight, pos_emb_cache, modality_embedding):
  bs, seq_len = inputs.shape
  emb_dim = shared_embed_weight.shape[1]

  x = _sc_embed_gather(
      shared_embed_weight, inputs.reshape(-1), modality_embedding)
  x = x.reshape(bs, seq_len, emb_dim)
  pos_emb = _tc_pos_emb(pos_emb_cache, bs)

  attn_pattern_mask = _tc_ones(bs * 4, seq_len).reshape(
      bs, 4, seq_len, seq_len)

  modality_id = jnp.array(0, dtype=jnp.int32)
  return (x, pos_emb, modality_id, attn_pattern_mask)


# speedup vs baseline: 1.0819x; 1.0819x over previous
import functools

import jax
import jax.numpy as jnp
from jax import lax
from jax.experimental import pallas as pl
from jax.experimental.pallas import tpu as pltpu
from jax.experimental.pallas import tpu_sc as plsc

_NC = 2
_NS = 16
_NW = _NC * _NS
_L = 16


def _sc_embed_gather(table, idx_flat, modality):
  V, D = table.shape
  B = idx_flat.shape[0]
  b_per_w = B // _NW
  R = 16
  NBUF = 3
  n_chunks = b_per_w // R
  mesh = plsc.VectorSubcoreMesh(core_axis_name="c", subcore_axis_name="s")

  @functools.partial(
      pl.kernel,
      mesh=mesh,
      out_type=jax.ShapeDtypeStruct((B, D), jnp.float32),
      scratch_types=[
          pltpu.VMEM((b_per_w,), jnp.int32),
          pltpu.VMEM((NBUF, R, D), jnp.float32),
          pltpu.VMEM((D,), jnp.float32),
          pltpu.SemaphoreType.DMA((NBUF,)),
          pltpu.SemaphoreType.DMA((NBUF,)),
      ],
  )
  def k(table_hbm, idx_hbm, mod_hbm, out_hbm,
        idx_v, rows_v, mod_v, gsem, osem):
    wid = lax.axis_index("s") * _NC + lax.axis_index("c")
    base = wid * b_per_w
    pltpu.sync_copy(idx_hbm.at[pl.ds(base, b_per_w)], idx_v)
    pltpu.sync_copy(mod_hbm, mod_v)

    def start_gather(c):
      b = c % NBUF
      return pltpu.async_copy(
          table_hbm.at[idx_v.at[pl.ds(c * R, R)]], rows_v.at[b],
          gsem.at[b])

    g = {c: start_gather(c) for c in range(min(NBUF - 1, n_chunks))}
    o = {}
    for c in range(n_chunks):
      b = c % NBUF
      g[c].wait()
      nxt = c + NBUF - 1
      if nxt < n_chunks:
        prev = nxt - NBUF
        if prev >= 0:
          o[prev].wait()
        g[nxt] = start_gather(nxt)

      UNROLL = 8

      def col_body(j, _):
        for u in range(UNROLL):
          sl = pl.ds((j * UNROLL + u) * _L, _L)
          m = mod_v[sl]
          for r in range(R):
            rows_v[b, r, sl] = rows_v[b, r, sl] + m
        return 0

      lax.fori_loop(0, D // (_L * UNROLL), col_body, 0)

      o[c] = pltpu.async_copy(
          rows_v.at[b], out_hbm.at[pl.ds(base + c * R, R)], osem.at[b])
    for c in range(max(0, n_chunks - NBUF), n_chunks):
      o[c].wait()

  return k(table, idx_flat, modality)


def _tc_pos_emb(cache, bs):
  S, D = cache.shape
  blk = 512

  def body(c_ref, o_ref):
    o_ref[...] = jnp.broadcast_to(c_ref[...][None], (bs, blk, D))

  return pl.pallas_call(
      body,
      grid=(S // blk,),
      in_specs=[pl.BlockSpec((blk, D), lambda i: (i, 0))],
      out_specs=pl.BlockSpec((bs, blk, D), lambda i: (0, i, 0)),
      out_shape=jax.ShapeDtypeStruct((bs, S, D), jnp.float32),
  )(cache)


def _tc_ones(rows, S):
  blk = 512

  def body(o_ref):
    o_ref[...] = jnp.ones_like(o_ref)

  return pl.pallas_call(
      body,
      grid=(rows, S // blk),
      out_specs=pl.BlockSpec((1, blk, S), lambda i, j: (i, j, 0)),
      out_shape=jax.ShapeDtypeStruct((rows, S, S), jnp.float32),
  )()


def kernel(inputs, shared_embed_weight, pos_emb_cache, modality_embedding):
  bs, seq_len = inputs.shape
  emb_dim = shared_embed_weight.shape[1]

  x = _sc_embed_gather(
      shared_embed_weight, inputs.reshape(-1), modality_embedding)
  x = x.reshape(bs, seq_len, emb_dim)
  pos_emb = _tc_pos_emb(pos_emb_cache, bs)

  attn_pattern_mask = _tc_ones(bs * 4, seq_len).reshape(
      bs, 4, seq_len, seq_len)

  modality_id = jnp.array(0, dtype=jnp.int32)
  return (x, pos_emb, modality_id, attn_pattern_mask)


# speedup vs baseline: 1.0908x; 1.0083x over previous
import functools

import jax
import jax.numpy as jnp
from jax import lax
from jax.experimental import pallas as pl
from jax.experimental.pallas import tpu as pltpu
from jax.experimental.pallas import tpu_sc as plsc

_NC = 2
_NS = 16
_NW = _NC * _NS
_L = 16


def _sc_embed_gather(table, idx2d, modality):
  V, D = table.shape
  bs, S = idx2d.shape
  B = bs * S
  b_per_w = B // _NW
  wpb = S // b_per_w
  R = 16
  NBUF = 3
  n_chunks = b_per_w // R
  mesh = plsc.VectorSubcoreMesh(core_axis_name="c", subcore_axis_name="s")

  @functools.partial(
      pl.kernel,
      mesh=mesh,
      out_type=jax.ShapeDtypeStruct((B, D), jnp.float32),
      scratch_types=[
          pltpu.VMEM((b_per_w,), jnp.int32),
          pltpu.VMEM((NBUF, R, D), jnp.float32),
          pltpu.VMEM((D,), jnp.float32),
          pltpu.SemaphoreType.DMA((NBUF,)),
          pltpu.SemaphoreType.DMA((NBUF,)),
      ],
  )
  def k(table_hbm, idx_hbm, mod_hbm, out_hbm,
        idx_v, rows_v, mod_v, gsem, osem):
    wid = lax.axis_index("s") * _NC + lax.axis_index("c")
    base = wid * b_per_w
    pltpu.sync_copy(
        idx_hbm.at[wid // wpb, pl.ds((wid % wpb) * b_per_w, b_per_w)], idx_v)
    pltpu.sync_copy(mod_hbm, mod_v)

    def start_gather(c):
      b = c % NBUF
      return pltpu.async_copy(
          table_hbm.at[idx_v.at[pl.ds(c * R, R)]], rows_v.at[b],
          gsem.at[b])

    g = {c: start_gather(c) for c in range(min(NBUF - 1, n_chunks))}
    o = {}
    for c in range(n_chunks):
      b = c % NBUF
      g[c].wait()
      nxt = c + NBUF - 1
      if nxt < n_chunks:
        prev = nxt - NBUF
        if prev >= 0:
          o[prev].wait()
        g[nxt] = start_gather(nxt)

      UNROLL = 8

      def col_body(j, _):
        for u in range(UNROLL):
          sl = pl.ds((j * UNROLL + u) * _L, _L)
          m = mod_v[sl]
          for r in range(R):
            rows_v[b, r, sl] = rows_v[b, r, sl] + m
        return 0

      lax.fori_loop(0, D // (_L * UNROLL), col_body, 0)

      o[c] = pltpu.async_copy(
          rows_v.at[b], out_hbm.at[pl.ds(base + c * R, R)], osem.at[b])
    for c in range(max(0, n_chunks - NBUF), n_chunks):
      o[c].wait()

  return k(table, idx2d, modality)


def _tc_pos_emb(cache, bs):
  S, D = cache.shape
  blk = 512

  def body(c_ref, o_ref):
    o_ref[...] = jnp.broadcast_to(c_ref[...][None], (bs, blk, D))

  return pl.pallas_call(
      body,
      grid=(S // blk,),
      in_specs=[pl.BlockSpec((blk, D), lambda i: (i, 0))],
      out_specs=pl.BlockSpec((bs, blk, D), lambda i: (0, i, 0)),
      out_shape=jax.ShapeDtypeStruct((bs, S, D), jnp.float32),
  )(cache)


def _tc_ones(rows, S):
  blk = 512

  def body(o_ref):
    o_ref[...] = jnp.ones_like(o_ref)

  return pl.pallas_call(
      body,
      grid=(rows, S // blk),
      out_specs=pl.BlockSpec((1, blk, S), lambda i, j: (i, j, 0)),
      out_shape=jax.ShapeDtypeStruct((rows, S, S), jnp.float32),
  )()


def kernel(inputs, shared_embed_weight, pos_emb_cache, modality_embedding):
  bs, seq_len = inputs.shape
  emb_dim = shared_embed_weight.shape[1]

  x = _sc_embed_gather(
      shared_embed_weight, inputs, modality_embedding)
  x = x.reshape(bs, seq_len, emb_dim)
  pos_emb = _tc_pos_emb(pos_emb_cache, bs)

  attn_pattern_mask = _tc_ones(bs * 4, seq_len).reshape(
      bs, 4, seq_len, seq_len)

  modality_id = jnp.array(0, dtype=jnp.int32)
  return (x, pos_emb, modality_id, attn_pattern_mask)
